# Initial kernel scaffold; baseline (speedup 1.0000x reference)
import jax
import jax.numpy as jnp
from jax.experimental import pallas as pl


def kernel(x, edge_index, W_l1, b_l1, W_r1, ln1_g, ln1_b, W_l2, b_l2, W_r2, ln2_g, ln2_b, W_p, b_p):
    raise NotImplementedError("write your pallas kernel here")


# speedup vs baseline: 3.0883x; 3.0883x over previous
import jax
import jax.numpy as jnp
from jax import lax
from jax.experimental import pallas as pl
from jax.experimental.pallas import tpu as pltpu
from jax.experimental.pallas import tpu_sc as plsc

NN = 10000
EE = 160000
F = 256
HF = 128
NS = 16
ET = EE // NS
CE = 80
NCH = ET // CE
NW = 10
RW = NN // NW
NZF = RW // CE
ZT = RW - NZF * CE
EW = EE // 32
CC = 40
NCC = EW // CC
NZC = RW // CC
RB = 1000
LN_EPS = 1e-5

_MESH = plsc.VectorSubcoreMesh(core_axis_name="c", subcore_axis_name="s")


def _sc_agg_body(table, src_hbm, dst_hbm, agg_out, srcb, dstb, rows,
                 acc, sem):
    c = lax.axis_index("c")
    s = lax.axis_index("s")
    zero16 = jnp.zeros((16,), jnp.float32)

    def zrow(i, carry):
        for j in range(HF // 16):
            rows[i, pl.ds(j * 16, 16)] = zero16
        return carry

    lax.fori_loop(0, CE, zrow, 0)

    @pl.when(s < NW)
    def _zero():
        for r in range(NZF):
            pltpu.sync_copy(rows, acc.at[pl.ds(s * RW + r * CE, CE)])
        pltpu.sync_copy(rows.at[pl.ds(0, ZT)],
                        acc.at[pl.ds(s * RW + NZF * CE, ZT)])

    plsc.subcore_barrier()

    off = c * NN

    def step(g, carry):
        base = s * ET + g * CE
        pltpu.sync_copy(src_hbm.at[pl.ds(base, CE)], srcb)
        pltpu.sync_copy(dst_hbm.at[pl.ds(base, CE)], dstb)
        for k in range(CE // 16):
            sl = pl.ds(k * 16, 16)
            srcb[sl] = srcb[sl] + off
        pltpu.async_copy(table.at[srcb], rows, sem).wait()
        pltpu.sync_copy(rows, acc.at[dstb], add=True)
        return carry

    lax.fori_loop(0, NCH, step, 0)

    plsc.subcore_barrier()

    @pl.when(s < NW)
    def _drain():
        ro = s * RW
        pltpu.sync_copy(acc.at[pl.ds(ro, RW)], agg_out.at[c, pl.ds(ro, RW)])


_sc_agg = pl.kernel(
    _sc_agg_body,
    mesh=_MESH,
    out_type=[jax.ShapeDtypeStruct((2, NN, HF), jnp.float32)],
    scratch_types=[
        pltpu.VMEM((CE,), jnp.int32),
        pltpu.VMEM((CE,), jnp.int32),
        pltpu.VMEM((CE, HF), jnp.float32),
        pltpu.VMEM_SHARED((NN, HF), jnp.float32),
        pltpu.SemaphoreType.DMA,
    ],
)


def _sc_count_body(dst_hbm, cnt_out, dstb, onesb, cacc):
    c = lax.axis_index("c")
    s = lax.axis_index("s")
    zero16 = jnp.zeros((16,), jnp.float32)
    one16 = jnp.ones((16,), jnp.float32)

    def zrow(i, carry):
        for j in range(HF // 16):
            onesb[i, pl.ds(j * 16, 16)] = zero16
        return carry

    lax.fori_loop(0, CC, zrow, 0)

    @pl.when(s < NW)
    def _zero():
        for r in range(NZC):
            pltpu.sync_copy(onesb, cacc.at[pl.ds(s * RW + r * CC, CC)])

    def orow(i, carry):
        for j in range(HF // 16):
            onesb[i, pl.ds(j * 16, 16)] = one16
        return carry

    lax.fori_loop(0, CC, orow, 0)
    plsc.subcore_barrier()

    wid = c * NS + s

    def step(g, carry):
        base = wid * EW + g * CC
        pltpu.sync_copy(dst_hbm.at[pl.ds(base, CC)], dstb)
        pltpu.sync_copy(onesb, cacc.at[dstb], add=True)
        return carry

    lax.fori_loop(0, NCC, step, 0)

    plsc.subcore_barrier()

    @pl.when(s < NW)
    def _drain():
        ro = s * RW
        pltpu.sync_copy(cacc.at[pl.ds(ro, RW)], cnt_out.at[c, pl.ds(ro, RW)])


_sc_count = pl.kernel(
    _sc_count_body,
    mesh=_MESH,
    out_type=[jax.ShapeDtypeStruct((2, NN, HF), jnp.float32)],
    scratch_types=[
        pltpu.VMEM((CC,), jnp.int32),
        pltpu.VMEM((CC, HF), jnp.float32),
        pltpu.VMEM_SHARED((NN, HF), jnp.float32),
    ],
)


def _mm_split(x_flat, W):
    def body(x_ref, w_ref, o_ref):
        y = jnp.dot(x_ref[...], w_ref[...],
                    preferred_element_type=jnp.float32,
                    precision=lax.Precision.HIGHEST)
        o_ref[0] = y[:, :HF]
        o_ref[1] = y[:, HF:]

    return pl.pallas_call(
        body,
        grid=(NN // RB,),
        in_specs=[pl.BlockSpec((RB, F), lambda i: (i, 0)),
                  pl.BlockSpec((F, F), lambda i: (0, 0))],
        out_specs=pl.BlockSpec((2, RB, HF), lambda i: (0, i, 0)),
        out_shape=jax.ShapeDtypeStruct((2, NN, HF), jnp.float32),
    )(x_flat, W)


def _sage_post(agg, cnt, xin, W_r, b_l, ln_g, ln_b, W_next):
    def body(agg_ref, cnt_ref, x_ref, wr_ref, bl_ref, g_ref, b_ref, wn_ref,
             h_ref, y_ref):
        aggc = jnp.concatenate([agg_ref[0], agg_ref[1]], axis=-1)
        total = cnt_ref[0][:, 0:1] + cnt_ref[1][:, 0:1]
        inv = 1.0 / jnp.maximum(total, 1.0)
        h = aggc * inv + bl_ref[...] + jnp.dot(
            x_ref[...], wr_ref[...], preferred_element_type=jnp.float32,
            precision=lax.Precision.HIGHEST)
        mu = jnp.mean(h, axis=-1, keepdims=True)
        var = jnp.mean(jnp.square(h - mu), axis=-1, keepdims=True)
        hn = (h - mu) * lax.rsqrt(var + LN_EPS) * g_ref[...] + b_ref[...]
        h1 = jnp.maximum(hn, 0.0)
        h_ref[...] = h1
        y = jnp.dot(h1, wn_ref[...], preferred_element_type=jnp.float32,
                    precision=lax.Precision.HIGHEST)
        y_ref[0] = y[:, :HF]
        y_ref[1] = y[:, HF:]

    return pl.pallas_call(
        body,
        grid=(NN // RB,),
        in_specs=[
            pl.BlockSpec((2, RB, HF), lambda i: (0, i, 0)),
            pl.BlockSpec((2, RB, HF), lambda i: (0, i, 0)),
            pl.BlockSpec((RB, F), lambda i: (i, 0)),
            pl.BlockSpec((F, F), lambda i: (0, 0)),
            pl.BlockSpec((1, F), lambda i: (0, 0)),
            pl.BlockSpec((1, F), lambda i: (0, 0)),
            pl.BlockSpec((1, F), lambda i: (0, 0)),
            pl.BlockSpec((F, F), lambda i: (0, 0)),
        ],
        out_specs=[pl.BlockSpec((RB, F), lambda i: (i, 0)),
                   pl.BlockSpec((2, RB, HF), lambda i: (0, i, 0))],
        out_shape=[jax.ShapeDtypeStruct((NN, F), jnp.float32),
                   jax.ShapeDtypeStruct((2, NN, HF), jnp.float32)],
    )(agg, cnt, xin, W_r, b_l, ln_g, ln_b, W_next)


def _final(agg, cnt, h1, W_r, b_l, ln_g, ln_b, W_p, b_p):
    def body(agg_ref, cnt_ref, h_ref, wr_ref, bl_ref, g_ref, b_ref,
             wp_ref, bp_ref, o_ref):
        aggc = jnp.concatenate([agg_ref[0], agg_ref[1]], axis=-1)
        total = cnt_ref[0][:, 0:1] + cnt_ref[1][:, 0:1]
        inv = 1.0 / jnp.maximum(total, 1.0)
        h = aggc * inv + bl_ref[...] + jnp.dot(
            h_ref[...], wr_ref[...], preferred_element_type=jnp.float32,
            precision=lax.Precision.HIGHEST)
        mu = jnp.mean(h, axis=-1, keepdims=True)
        var = jnp.mean(jnp.square(h - mu), axis=-1, keepdims=True)
        hn = (h - mu) * lax.rsqrt(var + LN_EPS) * g_ref[...] + b_ref[...]
        h2 = jnp.maximum(hn, 0.0)
        o_ref[...] = jnp.dot(h2, wp_ref[...],
                             preferred_element_type=jnp.float32,
                             precision=lax.Precision.HIGHEST) + bp_ref[...]

    return pl.pallas_call(
        body,
        grid=(NN // RB,),
        in_specs=[
            pl.BlockSpec((2, RB, HF), lambda i: (0, i, 0)),
            pl.BlockSpec((2, RB, HF), lambda i: (0, i, 0)),
            pl.BlockSpec((RB, F), lambda i: (i, 0)),
            pl.BlockSpec((F, F), lambda i: (0, 0)),
            pl.BlockSpec((1, F), lambda i: (0, 0)),
            pl.BlockSpec((1, F), lambda i: (0, 0)),
            pl.BlockSpec((1, F), lambda i: (0, 0)),
            pl.BlockSpec((F, F), lambda i: (0, 0)),
            pl.BlockSpec((1, F), lambda i: (0, 0)),
        ],
        out_specs=pl.BlockSpec((RB, F), lambda i: (i, 0)),
        out_shape=jax.ShapeDtypeStruct((NN, F), jnp.float32),
    )(agg, cnt, h1, W_r, b_l, ln_g, ln_b, W_p, b_p)


def kernel(x, edge_index, W_l1, b_l1, W_r1, ln1_g, ln1_b,
           W_l2, b_l2, W_r2, ln2_g, ln2_b, W_p, b_p):
    Bv, Nv, _ = x.shape
    x_flat = x.reshape(Bv * Nv, -1)
    offsets = (jnp.arange(Bv, dtype=edge_index.dtype) * Nv).reshape(Bv, 1, 1)
    ei = jnp.transpose(edge_index + offsets, (1, 0, 2)).reshape(2, -1)
    src, dst = ei[0], ei[1]

    b_l1r = b_l1.reshape(1, F)
    g1r = ln1_g.reshape(1, F)
    be1r = ln1_b.reshape(1, F)
    b_l2r = b_l2.reshape(1, F)
    g2r = ln2_g.reshape(1, F)
    be2r = ln2_b.reshape(1, F)
    b_pr = b_p.reshape(1, F)

    (cnt,) = _sc_count(dst)
    y1 = _mm_split(x_flat, W_l1).reshape(2 * NN, HF)
    (agg1,) = _sc_agg(y1, src, dst)
    h1, y2s = _sage_post(agg1, cnt, x_flat, W_r1, b_l1r, g1r, be1r, W_l2)
    y2 = y2s.reshape(2 * NN, HF)
    (agg2,) = _sc_agg(y2, src, dst)
    out = _final(agg2, cnt, h1, W_r2, b_l2r, g2r, be2r, W_p, b_pr)
    return out.reshape(Bv, Nv, F)


# speedup vs baseline: 3.2892x; 1.0651x over previous
import jax
import jax.numpy as jnp
from jax import lax
from jax.experimental import pallas as pl
from jax.experimental.pallas import tpu as pltpu
from jax.experimental.pallas import tpu_sc as plsc

NN = 10000
EE = 160000
F = 256
HF = 128
NS = 16
CE = 80
TR = 128
HR = 64
EP = TR * CE * NS
ER = EP // CE
NP = NN + 8
NW = 10
RW = NN // NW
NZF = RW // CE
ZT = RW - NZF * CE
CR = ER // 32
RB = 1000
LN_EPS = 1e-5

_MESH = plsc.VectorSubcoreMesh(core_axis_name="c", subcore_axis_name="s")


def _sc_agg_body(table, src_hbm, dst_hbm, agg_out, srcall, dstall,
                 rows0, rows1, acc, gs0, gs1, ss0, ss1):
    c = lax.axis_index("c")
    s = lax.axis_index("s")
    zero16 = jnp.zeros((16,), jnp.float32)

    def zrow(i, carry):
        for j in range(HF // 16):
            rows0[i, pl.ds(j * 16, 16)] = zero16
        return carry

    lax.fori_loop(0, CE, zrow, 0)

    @pl.when(s < NW)
    def _zero():
        for r in range(NZF):
            pltpu.sync_copy(rows0, acc.at[pl.ds(s * RW + r * CE, CE)])
        pltpu.sync_copy(rows0.at[pl.ds(0, ZT)],
                        acc.at[pl.ds(s * RW + NZF * CE, ZT)])

    plsc.subcore_barrier()

    off = c * NN

    def half(h, carry):
        base = s * TR + h * HR
        pltpu.sync_copy(src_hbm.at[pl.ds(base, HR)], srcall)
        pltpu.sync_copy(dst_hbm.at[pl.ds(base, HR)], dstall)

        def arow(i, c2):
            for k in range(CE // 16):
                sl = pl.ds(k * 16, 16)
                srcall[i, sl] = srcall[i, sl] + off
            return c2

        lax.fori_loop(0, HR, arow, 0)

        def pair(g0, c2):
            j0 = 2 * g0
            j1 = j0 + 1

            @pl.when(g0 > 0)
            def _w0():
                pltpu.make_async_copy(rows0, acc.at[dstall.at[j0]],
                                      ss0).wait()

            pltpu.async_copy(table.at[srcall.at[j0]], rows0, gs0)

            @pl.when(g0 > 0)
            def _w1():
                pltpu.make_async_copy(rows1, acc.at[dstall.at[j1]],
                                      ss1).wait()

            pltpu.async_copy(table.at[srcall.at[j1]], rows1, gs1)

            pltpu.make_async_copy(table.at[srcall.at[j0]], rows0, gs0).wait()
            pltpu.async_copy(rows0, acc.at[dstall.at[j0]], ss0, add=True)
            pltpu.make_async_copy(table.at[srcall.at[j1]], rows1, gs1).wait()
            pltpu.async_copy(rows1, acc.at[dstall.at[j1]], ss1, add=True)
            return c2

        lax.fori_loop(0, HR // 2, pair, 0)

        pltpu.make_async_copy(rows0, acc.at[dstall.at[0]], ss0).wait()
        pltpu.make_async_copy(rows1, acc.at[dstall.at[0]], ss1).wait()
        return carry

    lax.fori_loop(0, TR // HR, half, 0)

    plsc.subcore_barrier()

    @pl.when(s < NW)
    def _drain():
        ro = s * RW
        pltpu.sync_copy(acc.at[pl.ds(ro, RW)], agg_out.at[c, pl.ds(ro, RW)])


_sc_agg = pl.kernel(
    _sc_agg_body,
    mesh=_MESH,
    out_type=[jax.ShapeDtypeStruct((2, NN, HF), jnp.float32)],
    scratch_types=[
        pltpu.VMEM((HR, CE), jnp.int32),
        pltpu.VMEM((HR, CE), jnp.int32),
        pltpu.VMEM((CE, HF), jnp.float32),
        pltpu.VMEM((CE, HF), jnp.float32),
        pltpu.VMEM_SHARED((NP, HF), jnp.float32),
        pltpu.SemaphoreType.DMA,
        pltpu.SemaphoreType.DMA,
        pltpu.SemaphoreType.DMA,
        pltpu.SemaphoreType.DMA,
    ],
)


def _sc_count_body(dst_hbm, cnt_out, dstall, onesb, cacc, ss0, ss1):
    c = lax.axis_index("c")
    s = lax.axis_index("s")
    zero16 = jnp.zeros((16,), jnp.float32)
    one16 = jnp.ones((16,), jnp.float32)

    wid = c * NS + s
    pltpu.sync_copy(dst_hbm.at[pl.ds(wid * CR, CR)], dstall)

    def zrow(i, carry):
        for j in range(HF // 16):
            onesb[i, pl.ds(j * 16, 16)] = zero16
        return carry

    lax.fori_loop(0, CE, zrow, 0)

    @pl.when(s < NW)
    def _zero():
        for r in range(NZF):
            pltpu.sync_copy(onesb, cacc.at[pl.ds(s * RW + r * CE, CE)])
        pltpu.sync_copy(onesb.at[pl.ds(0, ZT)],
                        cacc.at[pl.ds(s * RW + NZF * CE, ZT)])

    def orow(i, carry):
        for j in range(HF // 16):
            onesb[i, pl.ds(j * 16, 16)] = one16
        return carry

    lax.fori_loop(0, CE, orow, 0)
    plsc.subcore_barrier()

    def pair(g0, carry):
        j0 = 2 * g0
        j1 = j0 + 1

        @pl.when(g0 > 0)
        def _w0():
            pltpu.make_async_copy(onesb, cacc.at[dstall.at[j0]], ss0).wait()

        pltpu.async_copy(onesb, cacc.at[dstall.at[j0]], ss0, add=True)

        @pl.when(g0 > 0)
        def _w1():
            pltpu.make_async_copy(onesb, cacc.at[dstall.at[j1]], ss1).wait()

        pltpu.async_copy(onesb, cacc.at[dstall.at[j1]], ss1, add=True)
        return carry

    lax.fori_loop(0, CR // 2, pair, 0)

    pltpu.make_async_copy(onesb, cacc.at[dstall.at[0]], ss0).wait()
    pltpu.make_async_copy(onesb, cacc.at[dstall.at[0]], ss1).wait()

    plsc.subcore_barrier()

    @pl.when(s < NW)
    def _drain():
        ro = s * RW
        pltpu.sync_copy(cacc.at[pl.ds(ro, RW)], cnt_out.at[c, pl.ds(ro, RW)])


_sc_count = pl.kernel(
    _sc_count_body,
    mesh=_MESH,
    out_type=[jax.ShapeDtypeStruct((2, NN, HF), jnp.float32)],
    scratch_types=[
        pltpu.VMEM((CR, CE), jnp.int32),
        pltpu.VMEM((CE, HF), jnp.float32),
        pltpu.VMEM_SHARED((NP, HF), jnp.float32),
        pltpu.SemaphoreType.DMA,
        pltpu.SemaphoreType.DMA,
    ],
)


def _mm_split(x_flat, W):
    def body(x_ref, w_ref, o_ref):
        y = jnp.dot(x_ref[...], w_ref[...],
                    preferred_element_type=jnp.float32,
                    precision=lax.Precision.HIGHEST)
        o_ref[0] = y[:, :HF]
        o_ref[1] = y[:, HF:]

    return pl.pallas_call(
        body,
        grid=(NN // RB,),
        in_specs=[pl.BlockSpec((RB, F), lambda i: (i, 0)),
                  pl.BlockSpec((F, F), lambda i: (0, 0))],
        out_specs=pl.BlockSpec((2, RB, HF), lambda i: (0, i, 0)),
        out_shape=jax.ShapeDtypeStruct((2, NN, HF), jnp.float32),
    )(x_flat, W)


def _sage_post(agg, cnt, xin, W_r, b_l, ln_g, ln_b, W_next):
    def body(agg_ref, cnt_ref, x_ref, wr_ref, bl_ref, g_ref, b_ref, wn_ref,
             h_ref, y_ref):
        aggc = jnp.concatenate([agg_ref[0], agg_ref[1]], axis=-1)
        total = cnt_ref[0][:, 0:1] + cnt_ref[1][:, 0:1]
        inv = 1.0 / jnp.maximum(total, 1.0)
        h = aggc * inv + bl_ref[...] + jnp.dot(
            x_ref[...], wr_ref[...], preferred_element_type=jnp.float32,
            precision=lax.Precision.HIGHEST)
        mu = jnp.mean(h, axis=-1, keepdims=True)
        var = jnp.mean(jnp.square(h - mu), axis=-1, keepdims=True)
        hn = (h - mu) * lax.rsqrt(var + LN_EPS) * g_ref[...] + b_ref[...]
        h1 = jnp.maximum(hn, 0.0)
        h_ref[...] = h1
        y = jnp.dot(h1, wn_ref[...], preferred_element_type=jnp.float32,
                    precision=lax.Precision.HIGHEST)
        y_ref[0] = y[:, :HF]
        y_ref[1] = y[:, HF:]

    return pl.pallas_call(
        body,
        grid=(NN // RB,),
        in_specs=[
            pl.BlockSpec((2, RB, HF), lambda i: (0, i, 0)),
            pl.BlockSpec((2, RB, HF), lambda i: (0, i, 0)),
            pl.BlockSpec((RB, F), lambda i: (i, 0)),
            pl.BlockSpec((F, F), lambda i: (0, 0)),
            pl.BlockSpec((1, F), lambda i: (0, 0)),
            pl.BlockSpec((1, F), lambda i: (0, 0)),
            pl.BlockSpec((1, F), lambda i: (0, 0)),
            pl.BlockSpec((F, F), lambda i: (0, 0)),
        ],
        out_specs=[pl.BlockSpec((RB, F), lambda i: (i, 0)),
                   pl.BlockSpec((2, RB, HF), lambda i: (0, i, 0))],
        out_shape=[jax.ShapeDtypeStruct((NN, F), jnp.float32),
                   jax.ShapeDtypeStruct((2, NN, HF), jnp.float32)],
    )(agg, cnt, xin, W_r, b_l, ln_g, ln_b, W_next)


def _final(agg, cnt, h1, W_r, b_l, ln_g, ln_b, W_p, b_p):
    def body(agg_ref, cnt_ref, h_ref, wr_ref, bl_ref, g_ref, b_ref,
             wp_ref, bp_ref, o_ref):
        aggc = jnp.concatenate([agg_ref[0], agg_ref[1]], axis=-1)
        total = cnt_ref[0][:, 0:1] + cnt_ref[1][:, 0:1]
        inv = 1.0 / jnp.maximum(total, 1.0)
        h = aggc * inv + bl_ref[...] + jnp.dot(
            h_ref[...], wr_ref[...], preferred_element_type=jnp.float32,
            precision=lax.Precision.HIGHEST)
        mu = jnp.mean(h, axis=-1, keepdims=True)
        var = jnp.mean(jnp.square(h - mu), axis=-1, keepdims=True)
        hn = (h - mu) * lax.rsqrt(var + LN_EPS) * g_ref[...] + b_ref[...]
        h2 = jnp.maximum(hn, 0.0)
        o_ref[...] = jnp.dot(h2, wp_ref[...],
                             preferred_element_type=jnp.float32,
                             precision=lax.Precision.HIGHEST) + bp_ref[...]

    return pl.pallas_call(
        body,
        grid=(NN // RB,),
        in_specs=[
            pl.BlockSpec((2, RB, HF), lambda i: (0, i, 0)),
            pl.BlockSpec((2, RB, HF), lambda i: (0, i, 0)),
            pl.BlockSpec((RB, F), lambda i: (i, 0)),
            pl.BlockSpec((F, F), lambda i: (0, 0)),
            pl.BlockSpec((1, F), lambda i: (0, 0)),
            pl.BlockSpec((1, F), lambda i: (0, 0)),
            pl.BlockSpec((1, F), lambda i: (0, 0)),
            pl.BlockSpec((F, F), lambda i: (0, 0)),
            pl.BlockSpec((1, F), lambda i: (0, 0)),
        ],
        out_specs=pl.BlockSpec((RB, F), lambda i: (i, 0)),
        out_shape=jax.ShapeDtypeStruct((NN, F), jnp.float32),
    )(agg, cnt, h1, W_r, b_l, ln_g, ln_b, W_p, b_p)


def kernel(x, edge_index, W_l1, b_l1, W_r1, ln1_g, ln1_b,
           W_l2, b_l2, W_r2, ln2_g, ln2_b, W_p, b_p):
    Bv, Nv, _ = x.shape
    x_flat = x.reshape(Bv * Nv, -1)
    offsets = (jnp.arange(Bv, dtype=edge_index.dtype) * Nv).reshape(Bv, 1, 1)
    ei = jnp.transpose(edge_index + offsets, (1, 0, 2)).reshape(2, -1)
    pad = EP - EE
    src = jnp.concatenate(
        [ei[0], jnp.zeros((pad,), ei.dtype)]).reshape(ER, CE)
    dst = jnp.concatenate(
        [ei[1], jnp.full((pad,), NN, ei.dtype)]).reshape(ER, CE)

    b_l1r = b_l1.reshape(1, F)
    g1r = ln1_g.reshape(1, F)
    be1r = ln1_b.reshape(1, F)
    b_l2r = b_l2.reshape(1, F)
    g2r = ln2_g.reshape(1, F)
    be2r = ln2_b.reshape(1, F)
    b_pr = b_p.reshape(1, F)

    (cnt,) = _sc_count(dst)
    y1 = _mm_split(x_flat, W_l1).reshape(2 * NN, HF)
    (agg1,) = _sc_agg(y1, src, dst)
    h1, y2s = _sage_post(agg1, cnt, x_flat, W_r1, b_l1r, g1r, be1r, W_l2)
    y2 = y2s.reshape(2 * NN, HF)
    (agg2,) = _sc_agg(y2, src, dst)
    out = _final(agg2, cnt, h1, W_r2, b_l2r, g2r, be2r, W_p, b_pr)
    return out.reshape(Bv, Nv, F)


# speedup vs baseline: 4.3005x; 1.3074x over previous
import jax
import jax.numpy as jnp
from jax import lax
from jax.experimental import pallas as pl
from jax.experimental.pallas import tpu as pltpu
from jax.experimental.pallas import tpu_sc as plsc

NB = 2500
EB = 40000
NN = 10000
EE = 160000
F = 256
HF = 128
NS = 16
CE = 128
EBP = 40960
RPB = EBP // CE
ER = 4 * RPB
TR = ER // NS
HR = 40
NP = NN + 8
NW = 10
RW = NN // NW
NZF = RW // CE
ZT = RW - NZF * CE
CR = ER // 32
RB = 1000
LN_EPS = 1e-5

_MESH = plsc.VectorSubcoreMesh(core_axis_name="c", subcore_axis_name="s")


def _sc_agg_body(table, src_hbm, dst_hbm, agg_out, srcall, dstall,
                 rows0, rows1, acc, gs0, gs1, ss0, ss1):
    c = lax.axis_index("c")
    s = lax.axis_index("s")
    zero16 = jnp.zeros((16,), jnp.float32)

    def zrow(i, carry):
        for j in range(HF // 16):
            rows0[i, pl.ds(j * 16, 16)] = zero16
        return carry

    lax.fori_loop(0, CE, zrow, 0)

    @pl.when(s < NW)
    def _zero():
        for r in range(NZF):
            pltpu.sync_copy(rows0, acc.at[pl.ds(s * RW + r * CE, CE)])
        pltpu.sync_copy(rows0.at[pl.ds(0, ZT)],
                        acc.at[pl.ds(s * RW + NZF * CE, ZT)])

    plsc.subcore_barrier()

    boff = (s // 4) * NB
    soff = boff + c * NN

    def half(h, carry):
        base = s * TR + h * HR
        pltpu.sync_copy(src_hbm.at[pl.ds(base, HR)], srcall)
        pltpu.sync_copy(dst_hbm.at[pl.ds(base, HR)], dstall)

        def arow(i, c2):
            for k in range(CE // 16):
                sl = pl.ds(k * 16, 16)
                srcall[i, sl] = srcall[i, sl] + soff
                dstall[i, sl] = dstall[i, sl] + boff
            return c2

        lax.fori_loop(0, HR, arow, 0)

        def pair(g0, c2):
            j0 = 2 * g0
            j1 = j0 + 1

            @pl.when(g0 > 0)
            def _w0():
                pltpu.make_async_copy(rows0, acc.at[dstall.at[j0]],
                                      ss0).wait()

            pltpu.async_copy(table.at[srcall.at[j0]], rows0, gs0)

            @pl.when(g0 > 0)
            def _w1():
                pltpu.make_async_copy(rows1, acc.at[dstall.at[j1]],
                                      ss1).wait()

            pltpu.async_copy(table.at[srcall.at[j1]], rows1, gs1)

            pltpu.make_async_copy(table.at[srcall.at[j0]], rows0, gs0).wait()
            pltpu.async_copy(rows0, acc.at[dstall.at[j0]], ss0, add=True)
            pltpu.make_async_copy(table.at[srcall.at[j1]], rows1, gs1).wait()
            pltpu.async_copy(rows1, acc.at[dstall.at[j1]], ss1, add=True)
            return c2

        lax.fori_loop(0, HR // 2, pair, 0)

        pltpu.make_async_copy(rows0, acc.at[dstall.at[0]], ss0).wait()
        pltpu.make_async_copy(rows1, acc.at[dstall.at[0]], ss1).wait()
        return carry

    lax.fori_loop(0, TR // HR, half, 0)

    plsc.subcore_barrier()

    @pl.when(s < NW)
    def _drain():
        ro = s * RW
        pltpu.sync_copy(acc.at[pl.ds(ro, RW)], agg_out.at[c, pl.ds(ro, RW)])


_sc_agg = pl.kernel(
    _sc_agg_body,
    mesh=_MESH,
    out_type=[jax.ShapeDtypeStruct((2, NN, HF), jnp.float32)],
    scratch_types=[
        pltpu.VMEM((HR, CE), jnp.int32),
        pltpu.VMEM((HR, CE), jnp.int32),
        pltpu.VMEM((CE, HF), jnp.float32),
        pltpu.VMEM((CE, HF), jnp.float32),
        pltpu.VMEM_SHARED((NP, HF), jnp.float32),
        pltpu.SemaphoreType.DMA,
        pltpu.SemaphoreType.DMA,
        pltpu.SemaphoreType.DMA,
        pltpu.SemaphoreType.DMA,
    ],
)


def _sc_count_body(dst_hbm, cnt_out, dstall, onesb, cacc, ss0, ss1):
    c = lax.axis_index("c")
    s = lax.axis_index("s")
    zero16 = jnp.zeros((16,), jnp.float32)
    one16 = jnp.ones((16,), jnp.float32)

    wid = c * NS + s
    pltpu.sync_copy(dst_hbm.at[pl.ds(wid * CR, CR)], dstall)
    boff = (wid // 8) * NB

    def arow(i, carry):
        for k in range(CE // 16):
            sl = pl.ds(k * 16, 16)
            dstall[i, sl] = dstall[i, sl] + boff
        return carry

    lax.fori_loop(0, CR, arow, 0)

    def zrow(i, carry):
        for j in range(HF // 16):
            onesb[i, pl.ds(j * 16, 16)] = zero16
        return carry

    lax.fori_loop(0, CE, zrow, 0)

    @pl.when(s < NW)
    def _zero():
        for r in range(NZF):
            pltpu.sync_copy(onesb, cacc.at[pl.ds(s * RW + r * CE, CE)])
        pltpu.sync_copy(onesb.at[pl.ds(0, ZT)],
                        cacc.at[pl.ds(s * RW + NZF * CE, ZT)])

    def orow(i, carry):
        for j in range(HF // 16):
            onesb[i, pl.ds(j * 16, 16)] = one16
        return carry

    lax.fori_loop(0, CE, orow, 0)
    plsc.subcore_barrier()

    def pair(g0, carry):
        j0 = 2 * g0
        j1 = j0 + 1

        @pl.when(g0 > 0)
        def _w0():
            pltpu.make_async_copy(onesb, cacc.at[dstall.at[j0]], ss0).wait()

        pltpu.async_copy(onesb, cacc.at[dstall.at[j0]], ss0, add=True)

        @pl.when(g0 > 0)
        def _w1():
            pltpu.make_async_copy(onesb, cacc.at[dstall.at[j1]], ss1).wait()

        pltpu.async_copy(onesb, cacc.at[dstall.at[j1]], ss1, add=True)
        return carry

    lax.fori_loop(0, CR // 2, pair, 0)

    pltpu.make_async_copy(onesb, cacc.at[dstall.at[0]], ss0).wait()
    pltpu.make_async_copy(onesb, cacc.at[dstall.at[0]], ss1).wait()

    plsc.subcore_barrier()

    @pl.when(s < NW)
    def _drain():
        ro = s * RW
        pltpu.sync_copy(cacc.at[pl.ds(ro, RW)], cnt_out.at[c, pl.ds(ro, RW)])


_sc_count = pl.kernel(
    _sc_count_body,
    mesh=_MESH,
    out_type=[jax.ShapeDtypeStruct((2, NN, HF), jnp.float32)],
    scratch_types=[
        pltpu.VMEM((CR, CE), jnp.int32),
        pltpu.VMEM((CE, HF), jnp.float32),
        pltpu.VMEM_SHARED((NP, HF), jnp.float32),
        pltpu.SemaphoreType.DMA,
        pltpu.SemaphoreType.DMA,
    ],
)


def _mm_split(x_flat, W):
    def body(x_ref, w_ref, o_ref):
        y = jnp.dot(x_ref[...], w_ref[...],
                    preferred_element_type=jnp.float32,
                    precision=lax.Precision.HIGHEST)
        o_ref[0] = y[:, :HF]
        o_ref[1] = y[:, HF:]

    return pl.pallas_call(
        body,
        grid=(NN // RB,),
        in_specs=[pl.BlockSpec((RB, F), lambda i: (i, 0)),
                  pl.BlockSpec((F, F), lambda i: (0, 0))],
        out_specs=pl.BlockSpec((2, RB, HF), lambda i: (0, i, 0)),
        out_shape=jax.ShapeDtypeStruct((2, NN, HF), jnp.float32),
    )(x_flat, W)


def _sage_post(agg, cnt, xin, W_r, b_l, ln_g, ln_b, W_next):
    def body(agg_ref, cnt_ref, x_ref, wr_ref, bl_ref, g_ref, b_ref, wn_ref,
             h_ref, y_ref):
        aggc = jnp.concatenate([agg_ref[0], agg_ref[1]], axis=-1)
        total = cnt_ref[0][:, 0:1] + cnt_ref[1][:, 0:1]
        inv = 1.0 / jnp.maximum(total, 1.0)
        h = aggc * inv + bl_ref[...] + jnp.dot(
            x_ref[...], wr_ref[...], preferred_element_type=jnp.float32,
            precision=lax.Precision.HIGHEST)
        mu = jnp.mean(h, axis=-1, keepdims=True)
        var = jnp.mean(jnp.square(h - mu), axis=-1, keepdims=True)
        hn = (h - mu) * lax.rsqrt(var + LN_EPS) * g_ref[...] + b_ref[...]
        h1 = jnp.maximum(hn, 0.0)
        h_ref[...] = h1
        y = jnp.dot(h1, wn_ref[...], preferred_element_type=jnp.float32,
                    precision=lax.Precision.HIGHEST)
        y_ref[0] = y[:, :HF]
        y_ref[1] = y[:, HF:]

    return pl.pallas_call(
        body,
        grid=(NN // RB,),
        in_specs=[
            pl.BlockSpec((2, RB, HF), lambda i: (0, i, 0)),
            pl.BlockSpec((2, RB, HF), lambda i: (0, i, 0)),
            pl.BlockSpec((RB, F), lambda i: (i, 0)),
            pl.BlockSpec((F, F), lambda i: (0, 0)),
            pl.BlockSpec((1, F), lambda i: (0, 0)),
            pl.BlockSpec((1, F), lambda i: (0, 0)),
            pl.BlockSpec((1, F), lambda i: (0, 0)),
            pl.BlockSpec((F, F), lambda i: (0, 0)),
        ],
        out_specs=[pl.BlockSpec((RB, F), lambda i: (i, 0)),
                   pl.BlockSpec((2, RB, HF), lambda i: (0, i, 0))],
        out_shape=[jax.ShapeDtypeStruct((NN, F), jnp.float32),
                   jax.ShapeDtypeStruct((2, NN, HF), jnp.float32)],
    )(agg, cnt, xin, W_r, b_l, ln_g, ln_b, W_next)


def _final(agg, cnt, h1, W_r, b_l, ln_g, ln_b, W_p, b_p):
    def body(agg_ref, cnt_ref, h_ref, wr_ref, bl_ref, g_ref, b_ref,
             wp_ref, bp_ref, o_ref):
        aggc = jnp.concatenate([agg_ref[0], agg_ref[1]], axis=-1)
        total = cnt_ref[0][:, 0:1] + cnt_ref[1][:, 0:1]
        inv = 1.0 / jnp.maximum(total, 1.0)
        h = aggc * inv + bl_ref[...] + jnp.dot(
            h_ref[...], wr_ref[...], preferred_element_type=jnp.float32,
            precision=lax.Precision.HIGHEST)
        mu = jnp.mean(h, axis=-1, keepdims=True)
        var = jnp.mean(jnp.square(h - mu), axis=-1, keepdims=True)
        hn = (h - mu) * lax.rsqrt(var + LN_EPS) * g_ref[...] + b_ref[...]
        h2 = jnp.maximum(hn, 0.0)
        o_ref[...] = jnp.dot(h2, wp_ref[...],
                             preferred_element_type=jnp.float32,
                             precision=lax.Precision.HIGHEST) + bp_ref[...]

    return pl.pallas_call(
        body,
        grid=(NN // RB,),
        in_specs=[
            pl.BlockSpec((2, RB, HF), lambda i: (0, i, 0)),
            pl.BlockSpec((2, RB, HF), lambda i: (0, i, 0)),
            pl.BlockSpec((RB, F), lambda i: (i, 0)),
            pl.BlockSpec((F, F), lambda i: (0, 0)),
            pl.BlockSpec((1, F), lambda i: (0, 0)),
            pl.BlockSpec((1, F), lambda i: (0, 0)),
            pl.BlockSpec((1, F), lambda i: (0, 0)),
            pl.BlockSpec((F, F), lambda i: (0, 0)),
            pl.BlockSpec((1, F), lambda i: (0, 0)),
        ],
        out_specs=pl.BlockSpec((RB, F), lambda i: (i, 0)),
        out_shape=jax.ShapeDtypeStruct((NN, F), jnp.float32),
    )(agg, cnt, h1, W_r, b_l, ln_g, ln_b, W_p, b_p)


def kernel(x, edge_index, W_l1, b_l1, W_r1, ln1_g, ln1_b,
           W_l2, b_l2, W_r2, ln2_g, ln2_b, W_p, b_p):
    Bv, Nv, _ = x.shape
    x_flat = x.reshape(Bv * Nv, -1)
    pad = EBP - EB
    spad = jnp.zeros((Bv, pad), edge_index.dtype)
    dpad = jnp.broadcast_to(
        (NN - jnp.arange(Bv, dtype=edge_index.dtype) * Nv)[:, None],
        (Bv, pad))
    src = jnp.concatenate([edge_index[:, 0, :], spad], axis=1).reshape(ER, CE)
    dst = jnp.concatenate([edge_index[:, 1, :], dpad], axis=1).reshape(ER, CE)

    b_l1r = b_l1.reshape(1, F)
    g1r = ln1_g.reshape(1, F)
    be1r = ln1_b.reshape(1, F)
    b_l2r = b_l2.reshape(1, F)
    g2r = ln2_g.reshape(1, F)
    be2r = ln2_b.reshape(1, F)
    b_pr = b_p.reshape(1, F)

    (cnt,) = _sc_count(dst)
    y1 = _mm_split(x_flat, W_l1).reshape(2 * NN, HF)
    (agg1,) = _sc_agg(y1, src, dst)
    h1, y2s = _sage_post(agg1, cnt, x_flat, W_r1, b_l1r, g1r, be1r, W_l2)
    y2 = y2s.reshape(2 * NN, HF)
    (agg2,) = _sc_agg(y2, src, dst)
    out = _final(agg2, cnt, h1, W_r2, b_l2r, g2r, be2r, W_p, b_pr)
    return out.reshape(Bv, Nv, F)


# speedup vs baseline: 5.3808x; 1.2512x over previous
import jax
import jax.numpy as jnp
from jax import lax
from jax.experimental import pallas as pl
from jax.experimental.pallas import tpu as pltpu
from jax.experimental.pallas import tpu_sc as plsc

NB = 2500
EB = 40000
NN = 10000
EE = 160000
F = 256
HF = 128
NS = 16
CE = 125
CF = CE // 16
RPB = EB // CE
TR = RPB // 4
HR = 40
NW = 10
RW = NN // NW
NZF = RW // CE
CR = RPB // 8
RB = 1000
LN_EPS = 1e-5

_MESH = plsc.VectorSubcoreMesh(core_axis_name="c", subcore_axis_name="s")


def _sc_agg_body(table, ei_hbm, agg_out, srcall, dstall,
                 rows0, rows1, acc, gs0, gs1, ss0, ss1):
    c = lax.axis_index("c")
    s = lax.axis_index("s")
    zero16 = jnp.zeros((16,), jnp.float32)
    lane = jnp.arange(16, dtype=jnp.int32)
    tail = lane >= (16 - (CE - CF * 16))

    def zrow(i, carry):
        for j in range(HF // 16):
            rows0[i, pl.ds(j * 16, 16)] = zero16
        return carry

    lax.fori_loop(0, CE, zrow, 0)

    @pl.when(s < NW)
    def _zero():
        for r in range(NZF):
            pltpu.sync_copy(rows0, acc.at[pl.ds(s * RW + r * CE, CE)])

    plsc.subcore_barrier()

    b = s // 4
    boff = b * NB
    soff = boff + c * NN
    srow = (2 * b) * RPB + (s - 4 * b) * TR
    drow = (2 * b + 1) * RPB + (s - 4 * b) * TR

    def _adj(ref, off):
        def arow(i, c2):
            for k in range(CF):
                sl = pl.ds(k * 16, 16)
                ref[i, sl] = ref[i, sl] + off
            sl = pl.ds(CE - 16, 16)
            v = ref[i, sl]
            ref[i, sl] = jnp.where(tail, v + off, v)
            return c2

        lax.fori_loop(0, HR, arow, 0)

    def half(h, carry):
        pltpu.sync_copy(ei_hbm.at[pl.ds(srow + h * HR, HR)], srcall)
        pltpu.sync_copy(ei_hbm.at[pl.ds(drow + h * HR, HR)], dstall)
        _adj(srcall, soff)
        _adj(dstall, boff)

        def pair(g0, c2):
            j0 = 2 * g0
            j1 = j0 + 1

            @pl.when(g0 > 0)
            def _w0():
                pltpu.make_async_copy(rows0, acc.at[dstall.at[j0]],
                                      ss0).wait()

            pltpu.async_copy(table.at[srcall.at[j0]], rows0, gs0)

            @pl.when(g0 > 0)
            def _w1():
                pltpu.make_async_copy(rows1, acc.at[dstall.at[j1]],
                                      ss1).wait()

            pltpu.async_copy(table.at[srcall.at[j1]], rows1, gs1)

            pltpu.make_async_copy(table.at[srcall.at[j0]], rows0, gs0).wait()
            pltpu.async_copy(rows0, acc.at[dstall.at[j0]], ss0, add=True)
            pltpu.make_async_copy(table.at[srcall.at[j1]], rows1, gs1).wait()
            pltpu.async_copy(rows1, acc.at[dstall.at[j1]], ss1, add=True)
            return c2

        lax.fori_loop(0, HR // 2, pair, 0)

        pltpu.make_async_copy(rows0, acc.at[dstall.at[0]], ss0).wait()
        pltpu.make_async_copy(rows1, acc.at[dstall.at[0]], ss1).wait()
        return carry

    lax.fori_loop(0, TR // HR, half, 0)

    plsc.subcore_barrier()

    @pl.when(s < NW)
    def _drain():
        ro = s * RW
        pltpu.sync_copy(acc.at[pl.ds(ro, RW)], agg_out.at[c, pl.ds(ro, RW)])


_sc_agg = pl.kernel(
    _sc_agg_body,
    mesh=_MESH,
    out_type=[jax.ShapeDtypeStruct((2, NN, HF), jnp.float32)],
    scratch_types=[
        pltpu.VMEM((HR, CE), jnp.int32),
        pltpu.VMEM((HR, CE), jnp.int32),
        pltpu.VMEM((CE, HF), jnp.float32),
        pltpu.VMEM((CE, HF), jnp.float32),
        pltpu.VMEM_SHARED((NN, HF), jnp.float32),
        pltpu.SemaphoreType.DMA,
        pltpu.SemaphoreType.DMA,
        pltpu.SemaphoreType.DMA,
        pltpu.SemaphoreType.DMA,
    ],
)


def _sc_count_body(ei_hbm, cnt_out, dstall, onesb, cacc, ss0, ss1):
    c = lax.axis_index("c")
    s = lax.axis_index("s")
    zero16 = jnp.zeros((16,), jnp.float32)
    one16 = jnp.ones((16,), jnp.float32)
    lane = jnp.arange(16, dtype=jnp.int32)
    tail = lane >= (16 - (CE - CF * 16))

    wid = c * NS + s
    b = wid // 8
    boff = b * NB
    drow = (2 * b + 1) * RPB + (wid - 8 * b) * CR
    pltpu.sync_copy(ei_hbm.at[pl.ds(drow, CR)], dstall)

    def arow(i, carry):
        for k in range(CF):
            sl = pl.ds(k * 16, 16)
            dstall[i, sl] = dstall[i, sl] + boff
        sl = pl.ds(CE - 16, 16)
        v = dstall[i, sl]
        dstall[i, sl] = jnp.where(tail, v + boff, v)
        return carry

    lax.fori_loop(0, CR, arow, 0)

    def zrow(i, carry):
        for j in range(HF // 16):
            onesb[i, pl.ds(j * 16, 16)] = zero16
        return carry

    lax.fori_loop(0, CE, zrow, 0)

    @pl.when(s < NW)
    def _zero():
        for r in range(NZF):
            pltpu.sync_copy(onesb, cacc.at[pl.ds(s * RW + r * CE, CE)])

    def orow(i, carry):
        for j in range(HF // 16):
            onesb[i, pl.ds(j * 16, 16)] = one16
        return carry

    lax.fori_loop(0, CE, orow, 0)
    plsc.subcore_barrier()

    def pair(g0, carry):
        j0 = 2 * g0
        j1 = j0 + 1

        @pl.when(g0 > 0)
        def _w0():
            pltpu.make_async_copy(onesb, cacc.at[dstall.at[j0]], ss0).wait()

        pltpu.async_copy(onesb, cacc.at[dstall.at[j0]], ss0, add=True)

        @pl.when(g0 > 0)
        def _w1():
            pltpu.make_async_copy(onesb, cacc.at[dstall.at[j1]], ss1).wait()

        pltpu.async_copy(onesb, cacc.at[dstall.at[j1]], ss1, add=True)
        return carry

    lax.fori_loop(0, CR // 2, pair, 0)

    pltpu.make_async_copy(onesb, cacc.at[dstall.at[0]], ss0).wait()
    pltpu.make_async_copy(onesb, cacc.at[dstall.at[0]], ss1).wait()

    plsc.subcore_barrier()

    @pl.when(s < NW)
    def _drain():
        ro = s * RW
        pltpu.sync_copy(cacc.at[pl.ds(ro, RW)], cnt_out.at[c, pl.ds(ro, RW)])


_sc_count = pl.kernel(
    _sc_count_body,
    mesh=_MESH,
    out_type=[jax.ShapeDtypeStruct((2, NN, HF), jnp.float32)],
    scratch_types=[
        pltpu.VMEM((CR, CE), jnp.int32),
        pltpu.VMEM((CE, HF), jnp.float32),
        pltpu.VMEM_SHARED((NN, HF), jnp.float32),
        pltpu.SemaphoreType.DMA,
        pltpu.SemaphoreType.DMA,
    ],
)


def _mm_split(x_flat, W):
    def body(x_ref, w_ref, o_ref):
        y = jnp.dot(x_ref[...], w_ref[...],
                    preferred_element_type=jnp.float32,
                    precision=lax.Precision.HIGHEST)
        o_ref[0] = y[:, :HF]
        o_ref[1] = y[:, HF:]

    return pl.pallas_call(
        body,
        grid=(NN // RB,),
        in_specs=[pl.BlockSpec((RB, F), lambda i: (i, 0)),
                  pl.BlockSpec((F, F), lambda i: (0, 0))],
        out_specs=pl.BlockSpec((2, RB, HF), lambda i: (0, i, 0)),
        out_shape=jax.ShapeDtypeStruct((2, NN, HF), jnp.float32),
    )(x_flat, W)


def _sage_post(agg, cnt, xin, W_r, b_l, ln_g, ln_b, W_next):
    def body(agg_ref, cnt_ref, x_ref, wr_ref, bl_ref, g_ref, b_ref, wn_ref,
             h_ref, y_ref):
        aggc = jnp.concatenate([agg_ref[0], agg_ref[1]], axis=-1)
        total = cnt_ref[0][:, 0:1] + cnt_ref[1][:, 0:1]
        inv = 1.0 / jnp.maximum(total, 1.0)
        h = aggc * inv + bl_ref[...] + jnp.dot(
            x_ref[...], wr_ref[...], preferred_element_type=jnp.float32,
            precision=lax.Precision.HIGHEST)
        mu = jnp.mean(h, axis=-1, keepdims=True)
        var = jnp.mean(jnp.square(h - mu), axis=-1, keepdims=True)
        hn = (h - mu) * lax.rsqrt(var + LN_EPS) * g_ref[...] + b_ref[...]
        h1 = jnp.maximum(hn, 0.0)
        h_ref[...] = h1
        y = jnp.dot(h1, wn_ref[...], preferred_element_type=jnp.float32,
                    precision=lax.Precision.HIGHEST)
        y_ref[0] = y[:, :HF]
        y_ref[1] = y[:, HF:]

    return pl.pallas_call(
        body,
        grid=(NN // RB,),
        in_specs=[
            pl.BlockSpec((2, RB, HF), lambda i: (0, i, 0)),
            pl.BlockSpec((2, RB, HF), lambda i: (0, i, 0)),
            pl.BlockSpec((RB, F), lambda i: (i, 0)),
            pl.BlockSpec((F, F), lambda i: (0, 0)),
            pl.BlockSpec((1, F), lambda i: (0, 0)),
            pl.BlockSpec((1, F), lambda i: (0, 0)),
            pl.BlockSpec((1, F), lambda i: (0, 0)),
            pl.BlockSpec((F, F), lambda i: (0, 0)),
        ],
        out_specs=[pl.BlockSpec((RB, F), lambda i: (i, 0)),
                   pl.BlockSpec((2, RB, HF), lambda i: (0, i, 0))],
        out_shape=[jax.ShapeDtypeStruct((NN, F), jnp.float32),
                   jax.ShapeDtypeStruct((2, NN, HF), jnp.float32)],
    )(agg, cnt, xin, W_r, b_l, ln_g, ln_b, W_next)


def _final(agg, cnt, h1, W_r, b_l, ln_g, ln_b, W_p, b_p):
    def body(agg_ref, cnt_ref, h_ref, wr_ref, bl_ref, g_ref, b_ref,
             wp_ref, bp_ref, o_ref):
        aggc = jnp.concatenate([agg_ref[0], agg_ref[1]], axis=-1)
        total = cnt_ref[0][:, 0:1] + cnt_ref[1][:, 0:1]
        inv = 1.0 / jnp.maximum(total, 1.0)
        h = aggc * inv + bl_ref[...] + jnp.dot(
            h_ref[...], wr_ref[...], preferred_element_type=jnp.float32,
            precision=lax.Precision.HIGHEST)
        mu = jnp.mean(h, axis=-1, keepdims=True)
        var = jnp.mean(jnp.square(h - mu), axis=-1, keepdims=True)
        hn = (h - mu) * lax.rsqrt(var + LN_EPS) * g_ref[...] + b_ref[...]
        h2 = jnp.maximum(hn, 0.0)
        o_ref[...] = jnp.dot(h2, wp_ref[...],
                             preferred_element_type=jnp.float32,
                             precision=lax.Precision.HIGHEST) + bp_ref[...]

    return pl.pallas_call(
        body,
        grid=(NN // RB,),
        in_specs=[
            pl.BlockSpec((2, RB, HF), lambda i: (0, i, 0)),
            pl.BlockSpec((2, RB, HF), lambda i: (0, i, 0)),
            pl.BlockSpec((RB, F), lambda i: (i, 0)),
            pl.BlockSpec((F, F), lambda i: (0, 0)),
            pl.BlockSpec((1, F), lambda i: (0, 0)),
            pl.BlockSpec((1, F), lambda i: (0, 0)),
            pl.BlockSpec((1, F), lambda i: (0, 0)),
            pl.BlockSpec((F, F), lambda i: (0, 0)),
            pl.BlockSpec((1, F), lambda i: (0, 0)),
        ],
        out_specs=pl.BlockSpec((RB, F), lambda i: (i, 0)),
        out_shape=jax.ShapeDtypeStruct((NN, F), jnp.float32),
    )(agg, cnt, h1, W_r, b_l, ln_g, ln_b, W_p, b_p)


def kernel(x, edge_index, W_l1, b_l1, W_r1, ln1_g, ln1_b,
           W_l2, b_l2, W_r2, ln2_g, ln2_b, W_p, b_p):
    Bv, Nv, _ = x.shape
    x_flat = x.reshape(Bv * Nv, -1)
    ei2 = edge_index.reshape(Bv * 2 * RPB, CE)

    b_l1r = b_l1.reshape(1, F)
    g1r = ln1_g.reshape(1, F)
    be1r = ln1_b.reshape(1, F)
    b_l2r = b_l2.reshape(1, F)
    g2r = ln2_g.reshape(1, F)
    be2r = ln2_b.reshape(1, F)
    b_pr = b_p.reshape(1, F)

    (cnt,) = _sc_count(ei2)
    y1 = _mm_split(x_flat, W_l1).reshape(2 * NN, HF)
    (agg1,) = _sc_agg(y1, ei2)
    h1, y2s = _sage_post(agg1, cnt, x_flat, W_r1, b_l1r, g1r, be1r, W_l2)
    y2 = y2s.reshape(2 * NN, HF)
    (agg2,) = _sc_agg(y2, ei2)
    out = _final(agg2, cnt, h1, W_r2, b_l2r, g2r, be2r, W_p, b_pr)
    return out.reshape(Bv, Nv, F)
